# Initial kernel scaffold; baseline (speedup 1.0000x reference)
import jax
import jax.numpy as jnp
from jax.experimental import pallas as pl


def kernel(x, table, W, b):
    raise NotImplementedError("write your pallas kernel here")


# speedup vs baseline: 1.6550x; 1.6550x over previous
import functools

import jax
import jax.numpy as jnp
from jax import lax
from jax.experimental import pallas as pl
from jax.experimental.pallas import tpu as pltpu
from jax.experimental.pallas import tpu_sc as plsc

_ROW_BLK = 8000


def _project_body(table_ref, w_ref, b_ref, out_ref):
    out_ref[...] = (
        jnp.dot(table_ref[...], w_ref[...], preferred_element_type=jnp.float32)
        + b_ref[0, 0]
    )


def _project(table, W, b):
    n, d = table.shape
    grid = n // _ROW_BLK
    t = pl.pallas_call(
        _project_body,
        grid=(grid,),
        in_specs=[
            pl.BlockSpec((_ROW_BLK, d), lambda i: (i, 0)),
            pl.BlockSpec((d, 1), lambda i: (0, 0)),
            pl.BlockSpec((1, 1), lambda i: (0, 0)),
        ],
        out_specs=pl.BlockSpec((_ROW_BLK, 1), lambda i: (i, 0)),
        out_shape=jax.ShapeDtypeStruct((n, 1), jnp.float32),
    )(table, W, b.reshape(1, 1))
    return t.reshape(n)


def _make_pool(nw, nc, rpw, hist):
    mesh = plsc.VectorSubcoreMesh(core_axis_name="c", subcore_axis_name="s")

    @functools.partial(
        pl.kernel,
        out_type=jax.ShapeDtypeStruct((nw * rpw,), jnp.float32),
        mesh=mesh,
        scratch_types=[
            pltpu.VMEM((hist * rpw,), jnp.int32),
            pltpu.VMEM((hist * rpw,), jnp.float32),
            pltpu.VMEM((rpw,), jnp.float32),
            pltpu.SemaphoreType.DMA,
        ],
    )
    def pool(t_hbm, idx_hbm, out_hbm, idx_v, vals_v, res_v, sem):
        wid = lax.axis_index("s") * nc + lax.axis_index("c")
        pltpu.sync_copy(idx_hbm.at[wid], idx_v)
        pltpu.async_copy(t_hbm.at[idx_v], vals_v, sem).wait()
        scale = 1.0 / hist
        for g in range(rpw // 16):
            def body(j, acc, g=g):
                return acc + vals_v[pl.ds(j * rpw + g * 16, 16)]
            acc = lax.fori_loop(0, hist, body, jnp.zeros((16,), jnp.float32))
            res_v[pl.ds(g * 16, 16)] = acc * scale
        pltpu.sync_copy(res_v, out_hbm.at[pl.ds(wid * rpw, rpw)])

    return pool


def kernel(x, table, W, b):
    batch, hist = x.shape
    info = plsc.get_sparse_core_info()
    nc, ns = info.num_cores, info.num_subcores
    nw = nc * ns
    rpw = batch // nw
    t = _project(table, W, b)
    idx = (
        x.astype(jnp.int32)
        .reshape(nw, rpw, hist)
        .transpose(0, 2, 1)
        .reshape(nw, hist * rpw)
    )
    y = _make_pool(nw, nc, rpw, hist)(t, idx)
    return y.reshape(batch, 1)


# speedup vs baseline: 2.5455x; 1.5380x over previous
import functools

import jax
import jax.numpy as jnp
from jax import lax
from jax.experimental import pallas as pl
from jax.experimental.pallas import tpu as pltpu
from jax.experimental.pallas import tpu_sc as plsc

_ROW_BLK = 8192


def _project_body(table_ref, w_ref, b_ref, out_ref):
    blk = table_ref.shape[0]
    d = table_ref.shape[1]
    w_rep = jnp.broadcast_to(w_ref[...], (d, 128))
    yw = jnp.dot(table_ref[...], w_rep, preferred_element_type=jnp.float32)
    yw3 = yw.reshape(blk // 128, 128, 128)
    eye = (
        lax.broadcasted_iota(jnp.int32, (128, 128), 0)
        == lax.broadcasted_iota(jnp.int32, (128, 128), 1)
    )
    sel = jnp.where(eye[None], yw3, 0.0)
    out_ref[...] = jnp.sum(sel, axis=1) + b_ref[0, 0]


def _project(table, W, b):
    n, d = table.shape
    grid = (n + _ROW_BLK - 1) // _ROW_BLK
    n_pad = grid * _ROW_BLK
    t = pl.pallas_call(
        _project_body,
        grid=(grid,),
        in_specs=[
            pl.BlockSpec((_ROW_BLK, d), lambda i: (i, 0)),
            pl.BlockSpec((d, 1), lambda i: (0, 0)),
            pl.BlockSpec((1, 1), lambda i: (0, 0)),
        ],
        out_specs=pl.BlockSpec((_ROW_BLK // 128, 128), lambda i: (i, 0)),
        out_shape=jax.ShapeDtypeStruct((n_pad // 128, 128), jnp.float32),
    )(table, W, b.reshape(1, 1))
    return t.reshape(n_pad)


def _make_pool(nw, nc, rpw, hist):
    mesh = plsc.VectorSubcoreMesh(core_axis_name="c", subcore_axis_name="s")

    @functools.partial(
        pl.kernel,
        out_type=jax.ShapeDtypeStruct((nw * rpw,), jnp.float32),
        mesh=mesh,
        scratch_types=[
            pltpu.VMEM((hist * rpw,), jnp.int32),
            pltpu.VMEM((hist * rpw,), jnp.float32),
            pltpu.VMEM((rpw,), jnp.float32),
            pltpu.SemaphoreType.DMA,
        ],
    )
    def pool(t_hbm, idx_hbm, out_hbm, idx_v, vals_v, res_v, sem):
        wid = lax.axis_index("s") * nc + lax.axis_index("c")
        pltpu.sync_copy(idx_hbm.at[wid], idx_v)
        pltpu.async_copy(t_hbm.at[idx_v], vals_v, sem).wait()
        scale = 1.0 / hist
        for g in range(rpw // 16):
            def body(j, acc, g=g):
                return acc + vals_v[pl.ds(j * rpw + g * 16, 16)]
            acc = lax.fori_loop(0, hist, body, jnp.zeros((16,), jnp.float32))
            res_v[pl.ds(g * 16, 16)] = acc * scale
        pltpu.sync_copy(res_v, out_hbm.at[pl.ds(wid * rpw, rpw)])

    return pool


def kernel(x, table, W, b):
    batch, hist = x.shape
    info = plsc.get_sparse_core_info()
    nc, ns = info.num_cores, info.num_subcores
    nw = nc * ns
    rpw = batch // nw
    t = _project(table, W, b)
    idx = (
        x.astype(jnp.int32)
        .reshape(nw, rpw, hist)
        .transpose(0, 2, 1)
        .reshape(nw, hist * rpw)
    )
    y = _make_pool(nw, nc, rpw, hist)(t, idx)
    return y.reshape(batch, 1)
